# Initial kernel scaffold; baseline (speedup 1.0000x reference)
import jax
import jax.numpy as jnp
from jax.experimental import pallas as pl


def kernel(x, edge_index, edge_attr, batch, x_emb1, x_emb2, lin_W, lin_b, ee1, ee2, bn_g, bn_b, feat_W, feat_b, p1_W, p1_b, p2_W, p2_b):
    raise NotImplementedError("write your pallas kernel here")


# speedup vs baseline: 9.6212x; 9.6212x over previous
import functools

import jax
import jax.numpy as jnp
from jax import lax
from jax.experimental import pallas as pl
from jax.experimental.pallas import tpu as pltpu
from jax.experimental.pallas import tpu_sc as plsc

_N = 10000
_E = 160000
_EMB = 256
_HALF = 128
_G = 256
_L = 5
_NC = 2
_NS = 16
_NPAD = 10240
_EPAD = 160256
_EPW = _EPAD // (_NC * _NS)
_EPS = _E // _NS
_EB = 200
_RB = 1000
_NB = _N // _RB
_F32 = jnp.float32
_HI = lax.Precision.HIGHEST

def _dot(a, b):
    return jax.lax.dot_general(a, b, (((1,), (0,)), ((), ())),
                               precision=_HI, preferred_element_type=_F32)


def _dott(a, b):
    return jax.lax.dot_general(a, b, (((0,), (0,)), ((), ())),
                               precision=_HI, preferred_element_type=_F32)


def _sc_deg_body(rows_hbm, z16_hbm, out_hbm, rowb, msg, acc):
    c = lax.axis_index("c")
    s = lax.axis_index("s")
    per = (_NPAD // 16) // _NS
    pltpu.sync_copy(z16_hbm.at[pl.ds(s * per, per)], acc.at[pl.ds(s * per, per)])
    wid = s * _NC + c
    pltpu.sync_copy(rows_hbm.at[pl.ds(wid * _EPW, _EPW)], rowb)
    plsc.subcore_barrier()
    i16 = lax.iota(jnp.int32, 16)

    def _step(k, _):
        r16 = rowb[pl.ds(k * 16, 16)]
        ri16 = lax.shift_right_logical(r16, 4)
        ci16 = jnp.bitwise_and(r16, 15)
        for i in range(16):
            msg[i, :] = (i16 == ci16[i]).astype(_F32)
        pltpu.sync_copy(msg, acc.at[ri16], add=True)
        return 0
    lax.fori_loop(0, _EPW // 16, _step, 0)

    plsc.subcore_barrier()
    pltpu.sync_copy(acc.at[pl.ds(s * per, per)],
                    out_hbm.at[c, pl.ds(s * per, per)])


def _sc_s_body(rows_hbm, cols_hbm, combos_hbm, dis_hbm, z16_hbm, out_hbm,
          rowb, colb, combob, disb, msg, acc):
    c = lax.axis_index("c")
    s = lax.axis_index("s")
    per = _NPAD // _NS
    pltpu.sync_copy(z16_hbm.at[pl.ds(s * per, per)], acc.at[pl.ds(s * per, per)])
    wid = s * _NC + c
    pltpu.sync_copy(rows_hbm.at[pl.ds(wid * _EPW, _EPW)], rowb)
    pltpu.sync_copy(cols_hbm.at[pl.ds(wid * _EPW, _EPW)], colb)
    pltpu.sync_copy(combos_hbm.at[pl.ds(wid * _EPW, _EPW)], combob)
    pltpu.sync_copy(dis_hbm, disb)
    plsc.subcore_barrier()
    i16 = lax.iota(jnp.int32, 16)

    def _step(k, _):
        base = k * 16
        c16 = colb[pl.ds(base, 16)]
        k16 = combob[pl.ds(base, 16)]
        r16 = rowb[pl.ds(base, 16)]
        nr16 = plsc.load_gather(disb, [r16])
        for i in range(16):
            msg[i, :] = jnp.where(i16 == k16[i], nr16[i], 0.0)
        pltpu.sync_copy(msg, acc.at[c16], add=True)
        return 0
    lax.fori_loop(0, _EPW // 16, _step, 0)

    plsc.subcore_barrier()
    pltpu.sync_copy(acc.at[pl.ds(s * per, per)],
                    out_hbm.at[c, pl.ds(s * per, per)])


def _sc_agg_body(hxs_hbm, row_hbm, rowp_hbm, col_hbm, z128_hbm, out_hbm,
                 idx_r, idx_c, rows, acc, sem):
    c = lax.axis_index("c")
    s = lax.axis_index("s")
    per = _NPAD // _NS
    pltpu.sync_copy(z128_hbm.at[pl.ds(s * per, per)],
                    acc.at[pl.ds(s * per, per)])
    plsc.subcore_barrier()
    base0 = s * _EPS

    def _blk(j, _):
        b = base0 + j * _EB

        @pl.when(c == 0)
        def _():
            pltpu.sync_copy(row_hbm.at[pl.ds(b, _EB)], idx_r)

        @pl.when(c == 1)
        def _():
            pltpu.sync_copy(rowp_hbm.at[pl.ds(b, _EB)], idx_r)

        pltpu.sync_copy(col_hbm.at[pl.ds(b, _EB)], idx_c)
        pltpu.async_copy(hxs_hbm.at[idx_r], rows, sem).wait()
        pltpu.sync_copy(rows, acc.at[idx_c], add=True)
        return 0
    lax.fori_loop(0, _EPS // _EB, _blk, 0)

    plsc.subcore_barrier()
    pltpu.sync_copy(acc.at[pl.ds(s * per, per)],
                    out_hbm.at[c, pl.ds(s * per, per)])


@functools.lru_cache(maxsize=None)
def _sc_kernels():
    mesh = plsc.VectorSubcoreMesh(
        core_axis_name="c", subcore_axis_name="s",
        num_cores=_NC, num_subcores=_NS)
    cp = pltpu.CompilerParams(needs_layout_passes=False)
    deg = pl.kernel(
        _sc_deg_body,
        out_type=jax.ShapeDtypeStruct((_NC, _NPAD // 16, 16), _F32),
        mesh=mesh,
        compiler_params=cp,
        scratch_types=[
            pltpu.VMEM((_EPW,), jnp.int32),
            pltpu.VMEM((16, 16), _F32),
            pltpu.VMEM_SHARED((_NPAD // 16, 16), _F32),
        ])
    smat = pl.kernel(
        _sc_s_body,
        out_type=jax.ShapeDtypeStruct((_NC, _NPAD, 16), _F32),
        mesh=mesh,
        compiler_params=cp,
        scratch_types=[
            pltpu.VMEM((_EPW,), jnp.int32),
            pltpu.VMEM((_EPW,), jnp.int32),
            pltpu.VMEM((_EPW,), jnp.int32),
            pltpu.VMEM((_NPAD,), _F32),
            pltpu.VMEM((16, 16), _F32),
            pltpu.VMEM_SHARED((_NPAD, 16), _F32),
        ])
    agg = pl.kernel(
        _sc_agg_body,
        out_type=jax.ShapeDtypeStruct((_NC, _NPAD, _HALF), _F32),
        mesh=mesh,
        compiler_params=cp,
        scratch_types=[
            pltpu.VMEM((_EB,), jnp.int32),
            pltpu.VMEM((_EB,), jnp.int32),
            pltpu.VMEM((_EB, _HALF), _F32),
            pltpu.VMEM_SHARED((_NPAD, _HALF), _F32),
            pltpu.SemaphoreType.DMA,
        ])
    return deg, smat, agg


def _embed_body(x0_ref, x1_ref, e1_ref, e2_ref, d0_ref, d1_ref, w_ref, b_ref,
                hxs_ref, dis_ref):
    x0 = x0_ref[0, 0, :]
    x1 = x1_ref[0, 0, :]
    oh0 = (x0[:, None] == lax.broadcasted_iota(jnp.int32, (_RB, 8), 1)
           ).astype(_F32)
    oh1 = (x1[:, None] == lax.broadcasted_iota(jnp.int32, (_RB, 8), 1)
           ).astype(_F32)
    h = _dot(oh0, e1_ref[...]) + _dot(oh1, e2_ref[...])
    deg = d0_ref[0, 0, :] + d1_ref[0, 0, :] + 1.0
    dis = lax.rsqrt(deg)
    dis_ref[0, 0, :] = dis
    hx = _dot(h, w_ref[...]) + b_ref[...]
    hxs = dis[:, None] * hx
    hxs_ref[0] = hxs[:, :_HALF]
    hxs_ref[1] = hxs[:, _HALF:]


def _tc_embed(x0r, x1r, e1, e2, d0r, d1r, w0, b0):
    return pl.pallas_call(
        _embed_body,
        grid=(_NB,),
        in_specs=[
            pl.BlockSpec((1, 1, _RB), lambda i: (i, 0, 0)),
            pl.BlockSpec((1, 1, _RB), lambda i: (i, 0, 0)),
            pl.BlockSpec((8, _EMB), lambda i: (0, 0)),
            pl.BlockSpec((8, _EMB), lambda i: (0, 0)),
            pl.BlockSpec((1, 1, _RB), lambda i: (i, 0, 0)),
            pl.BlockSpec((1, 1, _RB), lambda i: (i, 0, 0)),
            pl.BlockSpec((_EMB, _EMB), lambda i: (0, 0)),
            pl.BlockSpec((1, _EMB), lambda i: (0, 0)),
        ],
        out_specs=[
            pl.BlockSpec((2, _RB, _HALF), lambda i: (0, i, 0)),
            pl.BlockSpec((1, 1, _RB), lambda i: (i, 0, 0)),
        ],
        out_shape=[
            jax.ShapeDtypeStruct((2, _N, _HALF), _F32),
            jax.ShapeDtypeStruct((_NB, 1, _RB), _F32),
        ],
    )(x0r, x1r, e1, e2, d0r, d1r, w0, b0)


def _post_body(agg_ref, hxs_ref, sp_ref, tab_ref, dis_ref,
               z_ref, sum_ref, sumsq_ref):
    i = pl.program_id(0)
    aggf = jnp.concatenate([agg_ref[0], agg_ref[1]], axis=1)
    hxsf = jnp.concatenate([hxs_ref[0], hxs_ref[1]], axis=1)
    smat = sp_ref[0] + sp_ref[1]
    dis = dis_ref[0, 0, :]
    oh12 = (lax.broadcasted_iota(jnp.int32, (1, 16), 1) == 12).astype(_F32)
    smat = smat + dis[:, None] * oh12
    z = dis[:, None] * (aggf + hxsf + _dot(smat, tab_ref[...]))
    z_ref[...] = z

    @pl.when(i == 0)
    def _():
        sum_ref[...] = jnp.zeros_like(sum_ref)
        sumsq_ref[...] = jnp.zeros_like(sumsq_ref)

    sum_ref[...] += jnp.sum(z, axis=0, keepdims=True)
    sumsq_ref[...] += jnp.sum(z * z, axis=0, keepdims=True)


def _tc_post(agg, hxs, sp, tab, disr):
    return pl.pallas_call(
        _post_body,
        grid=(_NB,),
        in_specs=[
            pl.BlockSpec((2, _RB, _HALF), lambda i: (0, i, 0)),
            pl.BlockSpec((2, _RB, _HALF), lambda i: (0, i, 0)),
            pl.BlockSpec((2, _RB, 16), lambda i: (0, i, 0)),
            pl.BlockSpec((16, _EMB), lambda i: (0, 0)),
            pl.BlockSpec((1, 1, _RB), lambda i: (i, 0, 0)),
        ],
        out_specs=[
            pl.BlockSpec((_RB, _EMB), lambda i: (i, 0)),
            pl.BlockSpec((1, _EMB), lambda i: (0, 0)),
            pl.BlockSpec((1, _EMB), lambda i: (0, 0)),
        ],
        out_shape=[
            jax.ShapeDtypeStruct((_N, _EMB), _F32),
            jax.ShapeDtypeStruct((1, _EMB), _F32),
            jax.ShapeDtypeStruct((1, _EMB), _F32),
        ],
    )(agg, hxs, sp, tab, disr)


def _pre_body(z_ref, sum_ref, sumsq_ref, g_ref, bb_ref, w_ref, b_ref, dis_ref,
              hxs_ref):
    m = sum_ref[...] * (1.0 / _N)
    var = sumsq_ref[...] * (1.0 / _N) - m * m
    inv = lax.rsqrt(var + 1e-5)
    h = (z_ref[...] - m) * (inv * g_ref[...]) + bb_ref[...]
    h = jnp.maximum(h, 0.0)
    hx = _dot(h, w_ref[...]) + b_ref[...]
    hxs = dis_ref[0, 0, :][:, None] * hx
    hxs_ref[0] = hxs[:, :_HALF]
    hxs_ref[1] = hxs[:, _HALF:]


def _tc_pre(z, sums, sumsq, g, bb, w, b, disr):
    return pl.pallas_call(
        _pre_body,
        grid=(_NB,),
        in_specs=[
            pl.BlockSpec((_RB, _EMB), lambda i: (i, 0)),
            pl.BlockSpec((1, _EMB), lambda i: (0, 0)),
            pl.BlockSpec((1, _EMB), lambda i: (0, 0)),
            pl.BlockSpec((1, _EMB), lambda i: (0, 0)),
            pl.BlockSpec((1, _EMB), lambda i: (0, 0)),
            pl.BlockSpec((_EMB, _EMB), lambda i: (0, 0)),
            pl.BlockSpec((1, _EMB), lambda i: (0, 0)),
            pl.BlockSpec((1, 1, _RB), lambda i: (i, 0, 0)),
        ],
        out_specs=pl.BlockSpec((2, _RB, _HALF), lambda i: (0, i, 0)),
        out_shape=jax.ShapeDtypeStruct((2, _N, _HALF), _F32),
    )(z, sums, sumsq, g, bb, w, b, disr)


def _pool_body(z_ref, sum_ref, sumsq_ref, g_ref, bb_ref, batch_ref,
               fw_ref, fb_ref, p1w_ref, p1b_ref, p2w_ref, p2b_ref,
               hg_ref, out_ref, segsum, segcnt):
    i = pl.program_id(0)
    m = sum_ref[...] * (1.0 / _N)
    var = sumsq_ref[...] * (1.0 / _N) - m * m
    inv = lax.rsqrt(var + 1e-5)
    h = (z_ref[...] - m) * (inv * g_ref[...]) + bb_ref[...]
    b_ = batch_ref[0, 0, :]
    oh = (b_[:, None] == lax.broadcasted_iota(jnp.int32, (_RB, _G), 1)
          ).astype(_F32)

    @pl.when(i == 0)
    def _():
        segsum[...] = jnp.zeros_like(segsum)
        segcnt[...] = jnp.zeros_like(segcnt)

    segsum[...] += _dott(oh, h)
    segcnt[...] += _dott(oh, jnp.ones((_RB, 8), _F32))

    @pl.when(i == _NB - 1)
    def _():
        cnt = jnp.maximum(segcnt[:, :1], 1.0)
        hg = _dot(segsum[...] / cnt, fw_ref[...]) + fb_ref[...]
        hg_ref[...] = hg
        t = jnp.maximum(_dot(hg, p1w_ref[...]) + p1b_ref[...], 0.0)
        out_ref[...] = _dot(t, p2w_ref[...]) + p2b_ref[...]


def _tc_pool(z, sums, sumsq, g, bb, batchr, fw, fb, p1w, p1b, p2w, p2b):
    return pl.pallas_call(
        _pool_body,
        grid=(_NB,),
        in_specs=[
            pl.BlockSpec((_RB, _EMB), lambda i: (i, 0)),
            pl.BlockSpec((1, _EMB), lambda i: (0, 0)),
            pl.BlockSpec((1, _EMB), lambda i: (0, 0)),
            pl.BlockSpec((1, _EMB), lambda i: (0, 0)),
            pl.BlockSpec((1, _EMB), lambda i: (0, 0)),
            pl.BlockSpec((1, 1, _RB), lambda i: (i, 0, 0)),
            pl.BlockSpec((_EMB, _EMB), lambda i: (0, 0)),
            pl.BlockSpec((1, _EMB), lambda i: (0, 0)),
            pl.BlockSpec((_EMB, _EMB), lambda i: (0, 0)),
            pl.BlockSpec((1, _EMB), lambda i: (0, 0)),
            pl.BlockSpec((_EMB, _HALF), lambda i: (0, 0)),
            pl.BlockSpec((1, _HALF), lambda i: (0, 0)),
        ],
        out_specs=[
            pl.BlockSpec((_G, _EMB), lambda i: (0, 0)),
            pl.BlockSpec((_G, _HALF), lambda i: (0, 0)),
        ],
        out_shape=[
            jax.ShapeDtypeStruct((_G, _EMB), _F32),
            jax.ShapeDtypeStruct((_G, _HALF), _F32),
        ],
        scratch_shapes=[
            pltpu.VMEM((_G, _EMB), _F32),
            pltpu.VMEM((_G, 8), _F32),
        ],
    )(z, sums, sumsq, g, bb, batchr, fw, fb, p1w, p1b, p2w, p2b)


def kernel(x, edge_index, edge_attr, batch, x_emb1, x_emb2, lin_W, lin_b,
           ee1, ee2, bn_g, bn_b, feat_W, feat_b, p1_W, p1_b, p2_W, p2_b):
    f32 = _F32
    row = edge_index[0].astype(jnp.int32)
    col = edge_index[1].astype(jnp.int32)
    combo = (edge_attr[:, 0] * 3 + edge_attr[:, 1]).astype(jnp.int32)
    npad = _EPAD - _E
    trash = jnp.full((npad,), _NPAD - 1, jnp.int32)
    rowpad = jnp.concatenate([row, trash])
    colpad = jnp.concatenate([col, trash])
    combopad = jnp.concatenate([combo, jnp.full((npad,), 15, jnp.int32)])
    rowp = row + _N

    z16 = jnp.zeros((_NPAD // 16, 16), f32)
    z16b = jnp.zeros((_NPAD, 16), f32)
    z128 = jnp.zeros((_NPAD, _HALF), f32)

    x0r = x[:, 0].astype(jnp.int32).reshape(_NB, 1, _RB)
    x1r = x[:, 1].astype(jnp.int32).reshape(_NB, 1, _RB)
    e1 = x_emb1[:8].astype(f32)
    e2 = jnp.concatenate([x_emb2, jnp.zeros((5, _EMB), f32)], axis=0)
    batchr = batch.astype(jnp.int32).reshape(_NB, 1, _RB)

    tabs = (ee1[:, :, None, :] + ee2[:, None, :, :]).reshape(_L, 15, _EMB)
    tabs = jnp.concatenate([tabs, jnp.zeros((_L, 1, _EMB), f32)], axis=1)

    sc_deg, sc_s, sc_agg = _sc_kernels()
    deg2 = sc_deg(rowpad, z16)
    degf = (deg2[0] + deg2[1]).reshape(_NPAD)[:_N]
    d0r = deg2[0].reshape(_NPAD)[:_N].reshape(_NB, 1, _RB)
    d1r = deg2[1].reshape(_NPAD)[:_N].reshape(_NB, 1, _RB)
    del degf

    hxs, disr = _tc_embed(x0r, x1r, e1, e2, d0r, d1r,
                          lin_W[0], lin_b[0].reshape(1, _EMB))
    dis_padded = jnp.concatenate(
        [disr.reshape(_N), jnp.ones((_NPAD - _N,), f32)])

    sp = sc_s(rowpad, colpad, combopad, dis_padded, z16b)

    z = sums = sumsq = None
    for l in range(_L):
        agg = sc_agg(hxs.reshape(2 * _N, _HALF), row, rowp, col, z128)
        z, sums, sumsq = _tc_post(agg, hxs, sp, tabs[l], disr)
        if l < _L - 1:
            hxs = _tc_pre(z, sums, sumsq, bn_g[l].reshape(1, _EMB),
                          bn_b[l].reshape(1, _EMB), lin_W[l + 1],
                          lin_b[l + 1].reshape(1, _EMB), disr)

    hg, out = _tc_pool(z, sums, sumsq, bn_g[_L - 1].reshape(1, _EMB),
                       bn_b[_L - 1].reshape(1, _EMB), batchr,
                       feat_W, feat_b.reshape(1, _EMB),
                       p1_W, p1_b.reshape(1, _EMB),
                       p2_W, p2_b.reshape(1, _HALF))
    return (hg, out)


# speedup vs baseline: 11.4165x; 1.1866x over previous
import functools

import jax
import jax.numpy as jnp
from jax import lax
from jax.experimental import pallas as pl
from jax.experimental.pallas import tpu as pltpu
from jax.experimental.pallas import tpu_sc as plsc

_N = 10000
_E = 160000
_EMB = 256
_HALF = 128
_G = 256
_L = 5
_NC = 2
_NS = 16
_NPAD = 10240
_EPAD = 160256
_EPW = _EPAD // (_NC * _NS)
_EPS = _E // _NS
_EB = 80
_RB = 1000
_NB = _N // _RB
_F32 = jnp.float32
_HI = lax.Precision.HIGHEST

def _dot(a, b):
    return jax.lax.dot_general(a, b, (((1,), (0,)), ((), ())),
                               precision=_HI, preferred_element_type=_F32)


def _dott(a, b):
    return jax.lax.dot_general(a, b, (((0,), (0,)), ((), ())),
                               precision=_HI, preferred_element_type=_F32)


def _sc_deg_body(rows_hbm, z16_hbm, out_hbm, rowb, msg, acc):
    c = lax.axis_index("c")
    s = lax.axis_index("s")
    per = (_NPAD // 16) // _NS
    pltpu.sync_copy(z16_hbm.at[pl.ds(s * per, per)], acc.at[pl.ds(s * per, per)])
    wid = s * _NC + c
    pltpu.sync_copy(rows_hbm.at[pl.ds(wid * _EPW, _EPW)], rowb)
    plsc.subcore_barrier()
    i16 = lax.iota(jnp.int32, 16)

    def _step(k, _):
        r16 = rowb[pl.ds(k * 16, 16)]
        ri16 = lax.shift_right_logical(r16, 4)
        ci16 = jnp.bitwise_and(r16, 15)
        for i in range(16):
            msg[i, :] = (i16 == ci16[i]).astype(_F32)
        pltpu.sync_copy(msg, acc.at[ri16], add=True)
        return 0
    lax.fori_loop(0, _EPW // 16, _step, 0)

    plsc.subcore_barrier()
    pltpu.sync_copy(acc.at[pl.ds(s * per, per)],
                    out_hbm.at[c, pl.ds(s * per, per)])


def _sc_s_body(rows_hbm, cols_hbm, combos_hbm, dis_hbm, z16_hbm, out_hbm,
          rowb, colb, combob, disb, msg, acc):
    c = lax.axis_index("c")
    s = lax.axis_index("s")
    per = _NPAD // _NS
    pltpu.sync_copy(z16_hbm.at[pl.ds(s * per, per)], acc.at[pl.ds(s * per, per)])
    wid = s * _NC + c
    pltpu.sync_copy(rows_hbm.at[pl.ds(wid * _EPW, _EPW)], rowb)
    pltpu.sync_copy(cols_hbm.at[pl.ds(wid * _EPW, _EPW)], colb)
    pltpu.sync_copy(combos_hbm.at[pl.ds(wid * _EPW, _EPW)], combob)
    pltpu.sync_copy(dis_hbm, disb)
    plsc.subcore_barrier()
    i16 = lax.iota(jnp.int32, 16)

    def _step(k, _):
        base = k * 16
        c16 = colb[pl.ds(base, 16)]
        k16 = combob[pl.ds(base, 16)]
        r16 = rowb[pl.ds(base, 16)]
        nr16 = plsc.load_gather(disb, [r16])
        for i in range(16):
            msg[i, :] = jnp.where(i16 == k16[i], nr16[i], 0.0)
        pltpu.sync_copy(msg, acc.at[c16], add=True)
        return 0
    lax.fori_loop(0, _EPW // 16, _step, 0)

    plsc.subcore_barrier()
    pltpu.sync_copy(acc.at[pl.ds(s * per, per)],
                    out_hbm.at[c, pl.ds(s * per, per)])


def _sc_agg_body(hxs_hbm, row_hbm, rowp_hbm, col_hbm, z128_hbm, out_hbm,
                 ir, ic, rows0, rows1, acc, sg0, sg1, ss0, ss1):
    c = lax.axis_index("c")
    s = lax.axis_index("s")
    per = _NPAD // _NS
    nblk = _EPS // _EB
    npair = (nblk - 1) // 2
    pltpu.sync_copy(z128_hbm.at[pl.ds(s * per, per)],
                    acc.at[pl.ds(s * per, per)])
    base0 = s * _EPS

    @pl.when(c == 0)
    def _():
        pltpu.sync_copy(row_hbm.at[pl.ds(base0, _EPS)], ir)

    @pl.when(c == 1)
    def _():
        pltpu.sync_copy(rowp_hbm.at[pl.ds(base0, _EPS)], ir)

    pltpu.sync_copy(col_hbm.at[pl.ds(base0, _EPS)], ic)
    plsc.subcore_barrier()

    def gat(j, buf, sem):
        pltpu.async_copy(hxs_hbm.at[ir.at[pl.ds(j * _EB, _EB)]], buf, sem)

    def gat_wait(buf, sem):
        pltpu.make_async_copy(hxs_hbm.at[ir.at[pl.ds(0, _EB)]], buf,
                              sem).wait()

    def sca(j, buf, sem):
        pltpu.async_copy(buf, acc.at[ic.at[pl.ds(j * _EB, _EB)]], sem,
                         add=True)

    def sca_wait(buf, sem):
        pltpu.make_async_copy(buf, acc.at[ic.at[pl.ds(0, _EB)]], sem).wait()

    gat(0, rows0, sg0)

    def _blk(k, _):
        j0 = 2 * k
        gat_wait(rows0, sg0)
        gat(j0 + 1, rows1, sg1)
        sca(j0, rows0, ss0)
        gat_wait(rows1, sg1)
        sca(j0 + 1, rows1, ss1)

        @pl.when(k < npair - 1)
        def _():
            sca_wait(rows0, ss0)
            gat(j0 + 2, rows0, sg0)
            sca_wait(rows1, ss1)
        return 0
    lax.fori_loop(0, npair, _blk, 0)

    sca_wait(rows0, ss0)
    sca_wait(rows1, ss1)
    gat(nblk - 1, rows0, sg0)
    gat_wait(rows0, sg0)
    sca(nblk - 1, rows0, ss0)
    sca_wait(rows0, ss0)
    plsc.subcore_barrier()
    pltpu.sync_copy(acc.at[pl.ds(s * per, per)],
                    out_hbm.at[c, pl.ds(s * per, per)])


@functools.lru_cache(maxsize=None)
def _sc_kernels():
    mesh = plsc.VectorSubcoreMesh(
        core_axis_name="c", subcore_axis_name="s",
        num_cores=_NC, num_subcores=_NS)
    cp = pltpu.CompilerParams(needs_layout_passes=False)
    deg = pl.kernel(
        _sc_deg_body,
        out_type=jax.ShapeDtypeStruct((_NC, _NPAD // 16, 16), _F32),
        mesh=mesh,
        compiler_params=cp,
        scratch_types=[
            pltpu.VMEM((_EPW,), jnp.int32),
            pltpu.VMEM((16, 16), _F32),
            pltpu.VMEM_SHARED((_NPAD // 16, 16), _F32),
        ])
    smat = pl.kernel(
        _sc_s_body,
        out_type=jax.ShapeDtypeStruct((_NC, _NPAD, 16), _F32),
        mesh=mesh,
        compiler_params=cp,
        scratch_types=[
            pltpu.VMEM((_EPW,), jnp.int32),
            pltpu.VMEM((_EPW,), jnp.int32),
            pltpu.VMEM((_EPW,), jnp.int32),
            pltpu.VMEM((_NPAD,), _F32),
            pltpu.VMEM((16, 16), _F32),
            pltpu.VMEM_SHARED((_NPAD, 16), _F32),
        ])
    agg = pl.kernel(
        _sc_agg_body,
        out_type=jax.ShapeDtypeStruct((_NC, _NPAD, _HALF), _F32),
        mesh=mesh,
        compiler_params=cp,
        scratch_types=[
            pltpu.VMEM((_EPS,), jnp.int32),
            pltpu.VMEM((_EPS,), jnp.int32),
            pltpu.VMEM((_EB, _HALF), _F32),
            pltpu.VMEM((_EB, _HALF), _F32),
            pltpu.VMEM_SHARED((_NPAD, _HALF), _F32),
            pltpu.SemaphoreType.DMA,
            pltpu.SemaphoreType.DMA,
            pltpu.SemaphoreType.DMA,
            pltpu.SemaphoreType.DMA,
        ])
    return deg, smat, agg


def _embed_body(x0_ref, x1_ref, e1_ref, e2_ref, d0_ref, d1_ref, w_ref, b_ref,
                hxs_ref, dis_ref):
    x0 = x0_ref[0, 0, :]
    x1 = x1_ref[0, 0, :]
    oh0 = (x0[:, None] == lax.broadcasted_iota(jnp.int32, (_RB, 8), 1)
           ).astype(_F32)
    oh1 = (x1[:, None] == lax.broadcasted_iota(jnp.int32, (_RB, 8), 1)
           ).astype(_F32)
    h = _dot(oh0, e1_ref[...]) + _dot(oh1, e2_ref[...])
    deg = d0_ref[0, 0, :] + d1_ref[0, 0, :] + 1.0
    dis = lax.rsqrt(deg)
    dis_ref[0, 0, :] = dis
    hx = _dot(h, w_ref[...]) + b_ref[...]
    hxs = dis[:, None] * hx
    hxs_ref[0] = hxs[:, :_HALF]
    hxs_ref[1] = hxs[:, _HALF:]


def _tc_embed(x0r, x1r, e1, e2, d0r, d1r, w0, b0):
    return pl.pallas_call(
        _embed_body,
        grid=(_NB,),
        in_specs=[
            pl.BlockSpec((1, 1, _RB), lambda i: (i, 0, 0)),
            pl.BlockSpec((1, 1, _RB), lambda i: (i, 0, 0)),
            pl.BlockSpec((8, _EMB), lambda i: (0, 0)),
            pl.BlockSpec((8, _EMB), lambda i: (0, 0)),
            pl.BlockSpec((1, 1, _RB), lambda i: (i, 0, 0)),
            pl.BlockSpec((1, 1, _RB), lambda i: (i, 0, 0)),
            pl.BlockSpec((_EMB, _EMB), lambda i: (0, 0)),
            pl.BlockSpec((1, _EMB), lambda i: (0, 0)),
        ],
        out_specs=[
            pl.BlockSpec((2, _RB, _HALF), lambda i: (0, i, 0)),
            pl.BlockSpec((1, 1, _RB), lambda i: (i, 0, 0)),
        ],
        out_shape=[
            jax.ShapeDtypeStruct((2, _N, _HALF), _F32),
            jax.ShapeDtypeStruct((_NB, 1, _RB), _F32),
        ],
    )(x0r, x1r, e1, e2, d0r, d1r, w0, b0)


def _post_body(agg_ref, hxs_ref, sp_ref, tab_ref, dis_ref,
               z_ref, sum_ref, sumsq_ref):
    i = pl.program_id(0)
    aggf = jnp.concatenate([agg_ref[0], agg_ref[1]], axis=1)
    hxsf = jnp.concatenate([hxs_ref[0], hxs_ref[1]], axis=1)
    smat = sp_ref[0] + sp_ref[1]
    dis = dis_ref[0, 0, :]
    oh12 = (lax.broadcasted_iota(jnp.int32, (1, 16), 1) == 12).astype(_F32)
    smat = smat + dis[:, None] * oh12
    z = dis[:, None] * (aggf + hxsf + _dot(smat, tab_ref[...]))
    z_ref[...] = z

    @pl.when(i == 0)
    def _():
        sum_ref[...] = jnp.zeros_like(sum_ref)
        sumsq_ref[...] = jnp.zeros_like(sumsq_ref)

    sum_ref[...] += jnp.sum(z, axis=0, keepdims=True)
    sumsq_ref[...] += jnp.sum(z * z, axis=0, keepdims=True)


def _tc_post(agg, hxs, sp, tab, disr):
    return pl.pallas_call(
        _post_body,
        grid=(_NB,),
        in_specs=[
            pl.BlockSpec((2, _RB, _HALF), lambda i: (0, i, 0)),
            pl.BlockSpec((2, _RB, _HALF), lambda i: (0, i, 0)),
            pl.BlockSpec((2, _RB, 16), lambda i: (0, i, 0)),
            pl.BlockSpec((16, _EMB), lambda i: (0, 0)),
            pl.BlockSpec((1, 1, _RB), lambda i: (i, 0, 0)),
        ],
        out_specs=[
            pl.BlockSpec((_RB, _EMB), lambda i: (i, 0)),
            pl.BlockSpec((1, _EMB), lambda i: (0, 0)),
            pl.BlockSpec((1, _EMB), lambda i: (0, 0)),
        ],
        out_shape=[
            jax.ShapeDtypeStruct((_N, _EMB), _F32),
            jax.ShapeDtypeStruct((1, _EMB), _F32),
            jax.ShapeDtypeStruct((1, _EMB), _F32),
        ],
    )(agg, hxs, sp, tab, disr)


def _pre_body(z_ref, sum_ref, sumsq_ref, g_ref, bb_ref, w_ref, b_ref, dis_ref,
              hxs_ref):
    m = sum_ref[...] * (1.0 / _N)
    var = sumsq_ref[...] * (1.0 / _N) - m * m
    inv = lax.rsqrt(var + 1e-5)
    h = (z_ref[...] - m) * (inv * g_ref[...]) + bb_ref[...]
    h = jnp.maximum(h, 0.0)
    hx = _dot(h, w_ref[...]) + b_ref[...]
    hxs = dis_ref[0, 0, :][:, None] * hx
    hxs_ref[0] = hxs[:, :_HALF]
    hxs_ref[1] = hxs[:, _HALF:]


def _tc_pre(z, sums, sumsq, g, bb, w, b, disr):
    return pl.pallas_call(
        _pre_body,
        grid=(_NB,),
        in_specs=[
            pl.BlockSpec((_RB, _EMB), lambda i: (i, 0)),
            pl.BlockSpec((1, _EMB), lambda i: (0, 0)),
            pl.BlockSpec((1, _EMB), lambda i: (0, 0)),
            pl.BlockSpec((1, _EMB), lambda i: (0, 0)),
            pl.BlockSpec((1, _EMB), lambda i: (0, 0)),
            pl.BlockSpec((_EMB, _EMB), lambda i: (0, 0)),
            pl.BlockSpec((1, _EMB), lambda i: (0, 0)),
            pl.BlockSpec((1, 1, _RB), lambda i: (i, 0, 0)),
        ],
        out_specs=pl.BlockSpec((2, _RB, _HALF), lambda i: (0, i, 0)),
        out_shape=jax.ShapeDtypeStruct((2, _N, _HALF), _F32),
    )(z, sums, sumsq, g, bb, w, b, disr)


def _pool_body(z_ref, sum_ref, sumsq_ref, g_ref, bb_ref, batch_ref,
               fw_ref, fb_ref, p1w_ref, p1b_ref, p2w_ref, p2b_ref,
               hg_ref, out_ref, segsum, segcnt):
    i = pl.program_id(0)
    m = sum_ref[...] * (1.0 / _N)
    var = sumsq_ref[...] * (1.0 / _N) - m * m
    inv = lax.rsqrt(var + 1e-5)
    h = (z_ref[...] - m) * (inv * g_ref[...]) + bb_ref[...]
    b_ = batch_ref[0, 0, :]
    oh = (b_[:, None] == lax.broadcasted_iota(jnp.int32, (_RB, _G), 1)
          ).astype(_F32)

    @pl.when(i == 0)
    def _():
        segsum[...] = jnp.zeros_like(segsum)
        segcnt[...] = jnp.zeros_like(segcnt)

    segsum[...] += _dott(oh, h)
    segcnt[...] += _dott(oh, jnp.ones((_RB, 8), _F32))

    @pl.when(i == _NB - 1)
    def _():
        cnt = jnp.maximum(segcnt[:, :1], 1.0)
        hg = _dot(segsum[...] / cnt, fw_ref[...]) + fb_ref[...]
        hg_ref[...] = hg
        t = jnp.maximum(_dot(hg, p1w_ref[...]) + p1b_ref[...], 0.0)
        out_ref[...] = _dot(t, p2w_ref[...]) + p2b_ref[...]


def _tc_pool(z, sums, sumsq, g, bb, batchr, fw, fb, p1w, p1b, p2w, p2b):
    return pl.pallas_call(
        _pool_body,
        grid=(_NB,),
        in_specs=[
            pl.BlockSpec((_RB, _EMB), lambda i: (i, 0)),
            pl.BlockSpec((1, _EMB), lambda i: (0, 0)),
            pl.BlockSpec((1, _EMB), lambda i: (0, 0)),
            pl.BlockSpec((1, _EMB), lambda i: (0, 0)),
            pl.BlockSpec((1, _EMB), lambda i: (0, 0)),
            pl.BlockSpec((1, 1, _RB), lambda i: (i, 0, 0)),
            pl.BlockSpec((_EMB, _EMB), lambda i: (0, 0)),
            pl.BlockSpec((1, _EMB), lambda i: (0, 0)),
            pl.BlockSpec((_EMB, _EMB), lambda i: (0, 0)),
            pl.BlockSpec((1, _EMB), lambda i: (0, 0)),
            pl.BlockSpec((_EMB, _HALF), lambda i: (0, 0)),
            pl.BlockSpec((1, _HALF), lambda i: (0, 0)),
        ],
        out_specs=[
            pl.BlockSpec((_G, _EMB), lambda i: (0, 0)),
            pl.BlockSpec((_G, _HALF), lambda i: (0, 0)),
        ],
        out_shape=[
            jax.ShapeDtypeStruct((_G, _EMB), _F32),
            jax.ShapeDtypeStruct((_G, _HALF), _F32),
        ],
        scratch_shapes=[
            pltpu.VMEM((_G, _EMB), _F32),
            pltpu.VMEM((_G, 8), _F32),
        ],
    )(z, sums, sumsq, g, bb, batchr, fw, fb, p1w, p1b, p2w, p2b)


def kernel(x, edge_index, edge_attr, batch, x_emb1, x_emb2, lin_W, lin_b,
           ee1, ee2, bn_g, bn_b, feat_W, feat_b, p1_W, p1_b, p2_W, p2_b):
    f32 = _F32
    row = edge_index[0].astype(jnp.int32)
    col = edge_index[1].astype(jnp.int32)
    combo = (edge_attr[:, 0] * 3 + edge_attr[:, 1]).astype(jnp.int32)
    npad = _EPAD - _E
    trash = jnp.full((npad,), _NPAD - 1, jnp.int32)
    rowpad = jnp.concatenate([row, trash])
    colpad = jnp.concatenate([col, trash])
    combopad = jnp.concatenate([combo, jnp.full((npad,), 15, jnp.int32)])
    rowp = row + _N

    z16 = jnp.zeros((_NPAD // 16, 16), f32)
    z16b = jnp.zeros((_NPAD, 16), f32)
    z128 = jnp.zeros((_NPAD, _HALF), f32)

    x0r = x[:, 0].astype(jnp.int32).reshape(_NB, 1, _RB)
    x1r = x[:, 1].astype(jnp.int32).reshape(_NB, 1, _RB)
    e1 = x_emb1[:8].astype(f32)
    e2 = jnp.concatenate([x_emb2, jnp.zeros((5, _EMB), f32)], axis=0)
    batchr = batch.astype(jnp.int32).reshape(_NB, 1, _RB)

    tabs = (ee1[:, :, None, :] + ee2[:, None, :, :]).reshape(_L, 15, _EMB)
    tabs = jnp.concatenate([tabs, jnp.zeros((_L, 1, _EMB), f32)], axis=1)

    sc_deg, sc_s, sc_agg = _sc_kernels()
    deg2 = sc_deg(rowpad, z16)
    degf = (deg2[0] + deg2[1]).reshape(_NPAD)[:_N]
    d0r = deg2[0].reshape(_NPAD)[:_N].reshape(_NB, 1, _RB)
    d1r = deg2[1].reshape(_NPAD)[:_N].reshape(_NB, 1, _RB)
    del degf

    hxs, disr = _tc_embed(x0r, x1r, e1, e2, d0r, d1r,
                          lin_W[0], lin_b[0].reshape(1, _EMB))
    dis_padded = jnp.concatenate(
        [disr.reshape(_N), jnp.ones((_NPAD - _N,), f32)])

    sp = sc_s(rowpad, colpad, combopad, dis_padded, z16b)

    z = sums = sumsq = None
    for l in range(_L):
        agg = sc_agg(hxs.reshape(2 * _N, _HALF), row, rowp, col, z128)
        z, sums, sumsq = _tc_post(agg, hxs, sp, tabs[l], disr)
        if l < _L - 1:
            hxs = _tc_pre(z, sums, sumsq, bn_g[l].reshape(1, _EMB),
                          bn_b[l].reshape(1, _EMB), lin_W[l + 1],
                          lin_b[l + 1].reshape(1, _EMB), disr)

    hg, out = _tc_pool(z, sums, sumsq, bn_g[_L - 1].reshape(1, _EMB),
                       bn_b[_L - 1].reshape(1, _EMB), batchr,
                       feat_W, feat_b.reshape(1, _EMB),
                       p1_W, p1_b.reshape(1, _EMB),
                       p2_W, p2_b.reshape(1, _HALF))
    return (hg, out)
